# baseline (speedup 1.0000x reference)
import jax
import jax.numpy as jnp
from jax.experimental import pallas as pl
from jax.experimental.pallas import tpu as pltpu


def _body(x_ref, t_ref, h_ref, w_ref, o_ref):
    t = t_ref[0]
    h = h_ref[...]
    w = w_ref[...]
    wt = w + t
    pos = h[:, None, :] + wt[None, :, :]
    o_ref[0, 0] = x_ref[0, 0] + pos


def kernel(x, T_embed, H_embed, W_embed):
    B, T, H, W, C = x.shape
    grid = (B, T)
    return pl.pallas_call(
        _body,
        grid=grid,
        in_specs=[
            pl.BlockSpec((1, 1, H, W, C), lambda b, t: (b, t, 0, 0, 0)),
            pl.BlockSpec((1, 1, C), lambda b, t: (t, 0, 0)),
            pl.BlockSpec((H, C), lambda b, t: (0, 0)),
            pl.BlockSpec((W, C), lambda b, t: (0, 0)),
        ],
        out_specs=pl.BlockSpec((1, 1, H, W, C), lambda b, t: (b, t, 0, 0, 0)),
        out_shape=jax.ShapeDtypeStruct(x.shape, x.dtype),
        compiler_params=pltpu.CompilerParams(
            dimension_semantics=("parallel", "parallel"),
        ),
    )(x, T_embed[:T].reshape(T, 1, C), H_embed[:H], W_embed[:W])
